# baseline (speedup 1.0000x reference)
import functools

import jax
import jax.numpy as jnp
from jax import lax
from jax.experimental import pallas as pl
from jax.experimental.pallas import tpu as pltpu
from jax.experimental.pallas import tpu_sc as plsc

_NC = 2
_NS = 16
_NW = _NC * _NS
_CHUNK = 128


def _sc_gather(emb_item, emb_cate, emb_user, emb_age, emb_gender, emb_hour,
               emb_device, hist_item_idx, hist_cate_idx, small_idx, B, T, D):
    BT = B * T
    rw = BT // _NW
    nch = rw // _CHUNK
    nchp = ((nch + 7) // 8) * 8
    sb = B // _NW
    hidx3 = hist_item_idx.reshape(_NW, nch, _CHUNK)
    cidx3 = hist_cate_idx.reshape(_NW, nch, _CHUNK)
    pad = ((0, 0), (0, nchp - nch), (0, 0))
    hidx3 = jnp.pad(hidx3, pad)
    cidx3 = jnp.pad(cidx3, pad)
    small_idx = jnp.pad(small_idx, ((0, 0), (0, 1), (0, 0)))

    mesh = plsc.VectorSubcoreMesh(core_axis_name="c", subcore_axis_name="s")
    big = jax.ShapeDtypeStruct((BT, D), jnp.float32)
    sml = jax.ShapeDtypeStruct((B, D), jnp.float32)
    upad = jax.ShapeDtypeStruct((B, 4 * D), jnp.float32)

    @functools.partial(
        pl.kernel,
        mesh=mesh,
        compiler_params=pltpu.CompilerParams(use_tc_tiling_on_sc=False),
        out_type=[big, big, upad] + [sml] * 6,
        scratch_types=[
            pltpu.VMEM((nchp, _CHUNK), jnp.int32),
            pltpu.VMEM((nchp, _CHUNK), jnp.int32),
            pltpu.VMEM((8, _CHUNK), jnp.int32),
            pltpu.VMEM((_CHUNK, D), jnp.float32),
            pltpu.VMEM((_CHUNK, D), jnp.float32),
            pltpu.VMEM((_CHUNK, 4 * D), jnp.float32),
            pltpu.SemaphoreType.DMA,
            pltpu.SemaphoreType.DMA,
        ],
    )
    def gather_kernel(item_hbm, cate_hbm, user_hbm, age_hbm, gender_hbm,
                      hour_hbm, device_hbm, hidx_hbm, cidx_hbm, sidx_hbm,
                      hi_out, hc_out, u_out, a_out, g_out, ri_out, rc_out,
                      ho_out, de_out,
                      hidx_v, cidx_v, sidx_v, rows_a, rows_b, rows_u,
                      sem_a, sem_b):
        wid = lax.axis_index("s") * _NC + lax.axis_index("c")
        hbase = wid * rw
        sbase = wid * sb
        pltpu.sync_copy(hidx_hbm.at[wid], hidx_v)
        pltpu.sync_copy(cidx_hbm.at[wid], cidx_v)
        pltpu.sync_copy(sidx_hbm.at[wid], sidx_v)

        def chunk(j, carry):
            base = hbase + j * _CHUNK
            cpa = pltpu.async_copy(item_hbm.at[hidx_v.at[j]], rows_a, sem_a)
            cpb = pltpu.async_copy(cate_hbm.at[cidx_v.at[j]], rows_b, sem_b)
            cpa.wait()
            pltpu.sync_copy(rows_a, hi_out.at[pl.ds(base, _CHUNK)])
            cpb.wait()
            pltpu.sync_copy(rows_b, hc_out.at[pl.ds(base, _CHUNK)])
            return carry

        lax.fori_loop(0, nch, chunk, 0)

        def small(g, table, out_ref):
            pltpu.async_copy(table.at[sidx_v.at[g]], rows_a, sem_a).wait()
            pltpu.sync_copy(rows_a, out_ref.at[pl.ds(sbase, _CHUNK)])

        pltpu.async_copy(user_hbm.at[sidx_v.at[0]], rows_u, sem_a).wait()
        pltpu.sync_copy(rows_u, u_out.at[pl.ds(sbase, _CHUNK)])
        small(1, age_hbm, a_out)
        small(2, gender_hbm, g_out)
        small(3, item_hbm, ri_out)
        small(4, cate_hbm, rc_out)
        small(5, hour_hbm, ho_out)
        small(6, device_hbm, de_out)

    return gather_kernel(emb_item, emb_cate, emb_user, emb_age, emb_gender,
                         emb_hour, emb_device, hidx3, cidx3, small_idx)


def _dice_from_stats(z, s1, s2, n):
    mean = s1 * (1.0 / n)
    var = (s2 - n * mean * mean) * (1.0 / (n - 1))
    std = jnp.sqrt(jnp.maximum(var, 0.0))
    xn = (z - mean) / (std + 1e-8)
    p = jax.nn.sigmoid(xn)
    return z * (0.01 + 0.99 * p)


def _tca_body(hi_ref, hc_ref, ri_ref, rc_ref, u_ref, uoff_ref, a_ref, g_ref,
              ho_ref, de_ref, mask_ref, w1_ref, b1_ref, w2t_ref, b2_ref,
              mw1_ref, mb1_ref, z2_ref, stats2_ref, wacc_ref, *, btot):
    t = pl.program_id(0)
    hi = hi_ref[...]
    hc = hc_ref[...]
    h2 = jnp.concatenate([hi, hc], axis=-1)
    qv = jnp.concatenate([ri_ref[...], rc_ref[...]], axis=-1)
    w1 = w1_ref[...]
    d2 = h2.shape[-1]
    wh = w1[0:d2] - w1[2 * d2:3 * d2]
    wq = w1[d2:2 * d2] + w1[2 * d2:3 * d2]
    wd = w1[3 * d2:4 * d2]
    whd = jnp.concatenate([wh, wd], axis=0)
    hqh = jnp.concatenate([h2, h2 * qv], axis=-1)
    z = jnp.dot(hqh, whd, preferred_element_type=jnp.float32)
    z = z + jnp.dot(qv, wq, preferred_element_type=jnp.float32)
    z = z + b1_ref[...]
    s1 = jnp.sum(z, axis=0, keepdims=True)
    s2 = jnp.sum(z * z, axis=0, keepdims=True)
    act = _dice_from_stats(z, s1, s2, btot)
    scores = jnp.sum(act * w2t_ref[...], axis=1, keepdims=True)
    scores = (scores + b2_ref[0, 0]) * mask_ref[...]

    @pl.when(t == 0)
    def _():
        wacc_ref[...] = jnp.zeros_like(wacc_ref)

    wacc_ref[...] += scores * h2

    @pl.when(t == pl.num_programs(0) - 1)
    def _():
        d = hi.shape[-1]
        up = u_ref[...]
        uo = uoff_ref[...]
        ue = sum((uo == k).astype(jnp.float32) * up[:, k * d:(k + 1) * d]
                 for k in range(4))
        x = jnp.concatenate([ue, a_ref[...], g_ref[...], ho_ref[...],
                             de_ref[...], qv, wacc_ref[...]], axis=-1)
        z2 = jnp.dot(x, mw1_ref[...], preferred_element_type=jnp.float32)
        z2 = z2 + mb1_ref[...]
        z2_ref[...] = z2
        stats2_ref[0:1, :] = jnp.sum(z2, axis=0, keepdims=True)
        stats2_ref[1:2, :] = jnp.sum(z2 * z2, axis=0, keepdims=True)


def _tc3_body(z2_ref, stats2_ref, mw2_ref, mb2_ref, mw3_ref, mb3_ref,
              mw4_ref, mb4_ref, out_ref, *, btot):
    z2 = z2_ref[...]
    stats2 = stats2_ref[...]
    x = _dice_from_stats(z2, stats2[0][None], stats2[1][None], btot)

    def dice_full(z):
        s1 = jnp.sum(z, axis=0, keepdims=True)
        s2 = jnp.sum(z * z, axis=0, keepdims=True)
        return _dice_from_stats(z, s1, s2, btot)

    z3 = jnp.dot(x, mw2_ref[...], preferred_element_type=jnp.float32)
    x3 = dice_full(z3 + mb2_ref[...])
    z4 = jnp.dot(x3, mw3_ref[...], preferred_element_type=jnp.float32)
    x4 = dice_full(z4 + mb3_ref[...])
    logits = (x4[:, 0:1] * mw4_ref[0, 0] + x4[:, 1:2] * mw4_ref[1, 0]
              + mb4_ref[0, 0])
    out_ref[...] = jax.nn.sigmoid(logits)


def kernel(user_id, user_age, user_gender, recall_item_id, recall_cate_id,
           hist_item_id, hist_cate_id, ctx_hour, ctx_device, history_mask,
           params):
    p = params
    B, T = hist_item_id.shape
    D = p['emb_item_id'].shape[1]
    i32 = jnp.int32

    small_idx = jnp.stack([
        (user_id // 4).astype(i32), user_age.astype(i32),
        user_gender.astype(i32),
        recall_item_id.astype(i32), recall_cate_id.astype(i32),
        ctx_hour.astype(i32), ctx_device.astype(i32)], axis=0)
    small_idx = small_idx.reshape(7, _NW, B // _NW).transpose(1, 0, 2)
    uoff = (user_id % 4).astype(i32).reshape(B, 1)

    hi, hc, up, ae, ge, ri, rc, ho, de = _sc_gather(
        p['emb_item_id'], p['emb_cate_id'],
        p['emb_user_id'].reshape(-1, 4 * D),
        p['emb_user_age'], p['emb_user_gender'], p['emb_hour'],
        p['emb_device'],
        hist_item_id.astype(i32).T.reshape(-1),
        hist_cate_id.astype(i32).T.reshape(-1),
        small_idx, B, T, D)

    return _dense_forward(hi, hc, ri, rc, up, uoff, ae, ge, ho, de,
                          history_mask, p, T)


def _dense_forward(hi, hc, ri, rc, up, uoff, ae, ge, ho, de, history_mask,
                   p, T):
    B = ri.shape[0]
    D = ri.shape[1]
    NH = p['att_w1'].shape[1]

    b1 = p['att_b1'].reshape(1, NH)
    w2t = p['att_w2'].reshape(1, NH)
    b2 = p['att_b2'].reshape(1, 1)
    mb1 = p['mlp_b1'].reshape(1, -1)
    mb2 = p['mlp_b2'].reshape(1, -1)
    mb3 = p['mlp_b3'].reshape(1, -1)
    mb4 = p['mlp_b4'].reshape(1, 1)
    M1 = p['mlp_w1'].shape[1]
    M2 = p['mlp_w2'].shape[1]

    hblk = pl.BlockSpec((B, D), lambda t: (t, 0))
    cblk = pl.BlockSpec((B, D), lambda t: (0, 0))

    z2, stats2 = pl.pallas_call(
        functools.partial(_tca_body, btot=B),
        grid=(T,),
        in_specs=[
            hblk, hblk, cblk, cblk,
            pl.BlockSpec((B, 4 * D), lambda t: (0, 0)),
            pl.BlockSpec((B, 1), lambda t: (0, 0)),
            cblk, cblk, cblk, cblk,
            pl.BlockSpec((B, 1), lambda t: (t, 0)),
            pl.BlockSpec((8 * D, NH), lambda t: (0, 0)),
            pl.BlockSpec((1, NH), lambda t: (0, 0)),
            pl.BlockSpec((1, NH), lambda t: (0, 0)),
            pl.BlockSpec((1, 1), lambda t: (0, 0)),
            pl.BlockSpec((9 * D, M1), lambda t: (0, 0)),
            pl.BlockSpec((1, M1), lambda t: (0, 0)),
        ],
        out_specs=[
            pl.BlockSpec((B, M1), lambda t: (0, 0)),
            pl.BlockSpec((2, M1), lambda t: (0, 0)),
        ],
        out_shape=[
            jax.ShapeDtypeStruct((B, M1), jnp.float32),
            jax.ShapeDtypeStruct((2, M1), jnp.float32),
        ],
        scratch_shapes=[pltpu.VMEM((B, 2 * D), jnp.float32)],
    )(hi, hc, ri, rc, up, uoff, ae, ge, ho, de,
      history_mask.T.reshape(T * B, 1),
      p['att_w1'], b1, w2t, b2, p['mlp_w1'], mb1)

    out = pl.pallas_call(
        functools.partial(_tc3_body, btot=B),
        grid=(1,),
        in_specs=[
            pl.BlockSpec((B, M1), lambda i: (0, 0)),
            pl.BlockSpec((2, M1), lambda i: (0, 0)),
            pl.BlockSpec((M1, M2), lambda i: (0, 0)),
            pl.BlockSpec((1, M2), lambda i: (0, 0)),
            pl.BlockSpec((M2, 2), lambda i: (0, 0)),
            pl.BlockSpec((1, 2), lambda i: (0, 0)),
            pl.BlockSpec((2, 1), lambda i: (0, 0)),
            pl.BlockSpec((1, 1), lambda i: (0, 0)),
        ],
        out_specs=pl.BlockSpec((B, 1), lambda i: (0, 0)),
        out_shape=jax.ShapeDtypeStruct((B, 1), jnp.float32),
    )(z2, stats2, p['mlp_w2'], mb2, p['mlp_w3'], mb3, p['mlp_w4'], mb4)

    return out[:, 0]


# speedup vs baseline: 1.5891x; 1.5891x over previous
import functools

import jax
import jax.numpy as jnp
from jax import lax
from jax.experimental import pallas as pl
from jax.experimental.pallas import tpu as pltpu
from jax.experimental.pallas import tpu_sc as plsc

_NC = 2
_NS = 16
_NW = _NC * _NS
_CHUNK = 128


def _sc_gather(emb_item, emb_cate, emb_age, emb_gender, emb_hour,
               emb_device, hist_item_idx, hist_cate_idx, small_idx, B, T, D):
    BT = B * T
    rw = BT // _NW
    nch = rw // _CHUNK
    nchp = ((nch + 7) // 8) * 8
    sb = B // _NW
    hidx3 = hist_item_idx.reshape(_NW, nch, _CHUNK)
    cidx3 = hist_cate_idx.reshape(_NW, nch, _CHUNK)
    pad = ((0, 0), (0, nchp - nch), (0, 0))
    hidx3 = jnp.pad(hidx3, pad)
    cidx3 = jnp.pad(cidx3, pad)
    nsm = small_idx.shape[1]
    small_idx = jnp.pad(small_idx, ((0, 0), (0, 8 - nsm), (0, 0)))

    mesh = plsc.VectorSubcoreMesh(core_axis_name="c", subcore_axis_name="s")

    @functools.partial(
        pl.kernel,
        mesh=mesh,
        compiler_params=pltpu.CompilerParams(use_tc_tiling_on_sc=False),
        out_type=[
            jax.ShapeDtypeStruct((BT, 2 * D), jnp.float32),
            jax.ShapeDtypeStruct((B, 2 * D), jnp.float32),
            jax.ShapeDtypeStruct((B, 4 * D), jnp.float32),
        ],
        scratch_types=[
            pltpu.VMEM((nchp, _CHUNK), jnp.int32),
            pltpu.VMEM((nchp, _CHUNK), jnp.int32),
            pltpu.VMEM((8, _CHUNK), jnp.int32),
            pltpu.VMEM((_CHUNK, D), jnp.float32),
            pltpu.VMEM((_CHUNK, D), jnp.float32),
            pltpu.SemaphoreType.DMA,
            pltpu.SemaphoreType.DMA,
        ],
    )
    def gather_kernel(item_hbm, cate_hbm, age_hbm, gender_hbm,
                      hour_hbm, device_hbm, hidx_hbm, cidx_hbm, sidx_hbm,
                      h2_out, q_out, xs_out,
                      hidx_v, cidx_v, sidx_v, rows_a, rows_b, sem_a, sem_b):
        wid = lax.axis_index("s") * _NC + lax.axis_index("c")
        hbase = wid * rw
        sbase = wid * sb
        pltpu.sync_copy(hidx_hbm.at[wid], hidx_v)
        pltpu.sync_copy(cidx_hbm.at[wid], cidx_v)
        pltpu.sync_copy(sidx_hbm.at[wid], sidx_v)

        def chunk(j, carry):
            base = hbase + j * _CHUNK
            cpa = pltpu.async_copy(item_hbm.at[hidx_v.at[j]], rows_a, sem_a)
            cpb = pltpu.async_copy(cate_hbm.at[cidx_v.at[j]], rows_b, sem_b)
            cpa.wait()
            pltpu.sync_copy(rows_a,
                            h2_out.at[pl.ds(base, _CHUNK), pl.ds(0, D)])
            cpb.wait()
            pltpu.sync_copy(rows_b,
                            h2_out.at[pl.ds(base, _CHUNK), pl.ds(D, D)])
            return carry

        lax.fori_loop(0, nch, chunk, 0)

        def small(g, table, out_ref, col):
            pltpu.async_copy(table.at[sidx_v.at[g]], rows_a, sem_a).wait()
            pltpu.sync_copy(rows_a,
                            out_ref.at[pl.ds(sbase, _CHUNK), pl.ds(col, D)])

        small(0, age_hbm, xs_out, 0)
        small(1, gender_hbm, xs_out, D)
        small(2, item_hbm, q_out, 0)
        small(3, cate_hbm, q_out, D)
        small(4, hour_hbm, xs_out, 2 * D)
        small(5, device_hbm, xs_out, 3 * D)

    return gather_kernel(emb_item, emb_cate, emb_age, emb_gender,
                         emb_hour, emb_device, hidx3, cidx3, small_idx)


def _dice_from_stats(z, s1, s2, n):
    mean = s1 * (1.0 / n)
    var = (s2 - n * mean * mean) * (1.0 / (n - 1))
    std = jnp.sqrt(jnp.maximum(var, 0.0))
    xn = (z - mean) / (std + 1e-8)
    p = jax.nn.sigmoid(xn)
    return z * (0.01 + 0.99 * p)


def _tca_body(h2_ref, q_ref, u_ref, xs_ref, mask_ref, w1_ref, b1_ref,
              w2t_ref, b2_ref, mw1_ref, mb1_ref, mw2_ref, mb2_ref,
              mw3_ref, mb3_ref, mw4_ref, mb4_ref,
              out_ref, wacc_ref, qproj_ref, *, btot):
    t = pl.program_id(0)
    tb = h2_ref.shape[0] // btot
    h2 = h2_ref[...]
    qv = q_ref[...]
    w1 = w1_ref[...]
    d2 = h2.shape[-1]
    wh = w1[0:d2] - w1[2 * d2:3 * d2]
    wd = w1[3 * d2:4 * d2]
    whd = jnp.concatenate([wh, wd], axis=0)

    nh = w1.shape[-1]

    @pl.when(t == 0)
    def _():
        wacc_ref[...] = jnp.zeros_like(wacc_ref)
        wq = w1[d2:2 * d2] + w1[2 * d2:3 * d2]
        qp = jnp.dot(qv, wq, preferred_element_type=jnp.float32)
        qproj_ref[...] = jnp.concatenate([qp + b1_ref[...]] * tb, axis=-1)

    qvt = jnp.concatenate([qv] * tb, axis=0)
    hqh = jnp.concatenate([h2, h2 * qvt], axis=-1)
    zz = jnp.dot(hqh, whd, preferred_element_type=jnp.float32)
    z = jnp.concatenate([zz[k * btot:(k + 1) * btot] for k in range(tb)],
                        axis=-1)
    z = z + qproj_ref[...]
    s1 = jnp.sum(z, axis=0, keepdims=True)
    s2 = jnp.sum(z * z, axis=0, keepdims=True)
    act = _dice_from_stats(z, s1, s2, btot)
    sw = act * jnp.concatenate([w2t_ref[...]] * tb, axis=-1)
    acc = jnp.zeros((btot, h2.shape[-1]), jnp.float32)
    for k in range(tb):
        sk = jnp.sum(sw[:, k * nh:(k + 1) * nh], axis=1, keepdims=True)
        sk = (sk + b2_ref[0, 0]) * mask_ref[k * btot:(k + 1) * btot]
        acc = acc + sk * h2[k * btot:(k + 1) * btot]
    wacc_ref[...] += acc

    @pl.when(t == pl.num_programs(0) - 1)
    def _():
        def dice_full(z):
            a = jnp.sum(z, axis=0, keepdims=True)
            b = jnp.sum(z * z, axis=0, keepdims=True)
            return _dice_from_stats(z, a, b, btot)

        x = jnp.concatenate([u_ref[...], xs_ref[...], qv, wacc_ref[...]],
                            axis=-1)
        z2 = jnp.dot(x, mw1_ref[...], preferred_element_type=jnp.float32)
        x2 = dice_full(z2 + mb1_ref[...])
        z3 = jnp.dot(x2, mw2_ref[...], preferred_element_type=jnp.float32)
        x3 = dice_full(z3 + mb2_ref[...])
        z4 = jnp.dot(x3, mw3_ref[...], preferred_element_type=jnp.float32)
        x4 = dice_full(z4 + mb3_ref[...])
        logits = (x4[:, 0:1] * mw4_ref[0, 0] + x4[:, 1:2] * mw4_ref[1, 0]
                  + mb4_ref[0, 0])
        out_ref[...] = jax.nn.sigmoid(logits)


def kernel(user_id, user_age, user_gender, recall_item_id, recall_cate_id,
           hist_item_id, hist_cate_id, ctx_hour, ctx_device, history_mask,
           params):
    p = params
    B, T = hist_item_id.shape
    D = p['emb_item_id'].shape[1]
    i32 = jnp.int32

    small_idx = jnp.stack([
        user_age.astype(i32), user_gender.astype(i32),
        recall_item_id.astype(i32), recall_cate_id.astype(i32),
        ctx_hour.astype(i32), ctx_device.astype(i32)], axis=0)
    small_idx = small_idx.reshape(6, _NW, B // _NW).transpose(1, 0, 2)

    h2, q, xs = _sc_gather(
        p['emb_item_id'], p['emb_cate_id'],
        p['emb_user_age'], p['emb_user_gender'], p['emb_hour'],
        p['emb_device'],
        hist_item_id.astype(i32).T.reshape(-1),
        hist_cate_id.astype(i32).T.reshape(-1),
        small_idx, B, T, D)

    ue = p['emb_user_id'][user_id]

    return _dense_forward(h2, q, ue, xs, history_mask, p, T)


def _dense_forward(h2, q, ue, xs, history_mask, p, T):
    B = q.shape[0]
    D = q.shape[1] // 2
    NH = p['att_w1'].shape[1]

    b1 = p['att_b1'].reshape(1, NH)
    w2t = p['att_w2'].reshape(1, NH)
    b2 = p['att_b2'].reshape(1, 1)
    mb1 = p['mlp_b1'].reshape(1, -1)
    mb2 = p['mlp_b2'].reshape(1, -1)
    mb3 = p['mlp_b3'].reshape(1, -1)
    mb4 = p['mlp_b4'].reshape(1, 1)
    M1 = p['mlp_w1'].shape[1]
    M2 = p['mlp_w2'].shape[1]

    TB = 2
    out = pl.pallas_call(
        functools.partial(_tca_body, btot=B),
        grid=(T // TB,),
        in_specs=[
            pl.BlockSpec((TB * B, 2 * D), lambda t: (t, 0)),
            pl.BlockSpec((B, 2 * D), lambda t: (0, 0)),
            pl.BlockSpec((B, D), lambda t: (0, 0)),
            pl.BlockSpec((B, 4 * D), lambda t: (0, 0)),
            pl.BlockSpec((TB * B, 1), lambda t: (t, 0)),
            pl.BlockSpec((8 * D, NH), lambda t: (0, 0)),
            pl.BlockSpec((1, NH), lambda t: (0, 0)),
            pl.BlockSpec((1, NH), lambda t: (0, 0)),
            pl.BlockSpec((1, 1), lambda t: (0, 0)),
            pl.BlockSpec((9 * D, M1), lambda t: (0, 0)),
            pl.BlockSpec((1, M1), lambda t: (0, 0)),
            pl.BlockSpec((M1, M2), lambda t: (0, 0)),
            pl.BlockSpec((1, M2), lambda t: (0, 0)),
            pl.BlockSpec((M2, 2), lambda t: (0, 0)),
            pl.BlockSpec((1, 2), lambda t: (0, 0)),
            pl.BlockSpec((2, 1), lambda t: (0, 0)),
            pl.BlockSpec((1, 1), lambda t: (0, 0)),
        ],
        out_specs=pl.BlockSpec((B, 1), lambda t: (0, 0)),
        out_shape=jax.ShapeDtypeStruct((B, 1), jnp.float32),
        scratch_shapes=[pltpu.VMEM((B, 2 * D), jnp.float32),
                        pltpu.VMEM((B, TB * NH), jnp.float32)],
    )(h2, q, ue, xs,
      history_mask.T.reshape(T * B, 1),
      p['att_w1'], b1, w2t, b2, p['mlp_w1'], mb1,
      p['mlp_w2'], mb2, p['mlp_w3'], mb3, p['mlp_w4'], mb4)

    return out[:, 0]
